# baseline (speedup 1.0000x reference)
import functools

import jax
import jax.numpy as jnp
from jax import lax
from jax.experimental import pallas as pl
from jax.experimental.pallas import tpu as pltpu
from jax.experimental.pallas import tpu_sc as plsc

B = 4
S = 8192
D = 1024
NC = 2
NS = 16
NW = NC * NS
SEQ_PER_W = S // NW
CH = 16
NCHUNK = SEQ_PER_W // CH
LANES = 16
VECS = (CH * D) // LANES
SC_B = 3


def _sc_kernel(x_hbm, pe_hbm, out_hbm, xbuf, pebuf, xsem, psem, osem):
    wid = lax.axis_index("s") * NC + lax.axis_index("c")
    s0 = wid * SEQ_PER_W

    def pe_load(c, slot):
        pltpu.async_copy(
            pe_hbm.at[pl.ds(s0 + c * CH, CH)], pebuf.at[slot], psem.at[slot])

    def pe_wait(c, slot):
        pltpu.make_async_copy(
            pe_hbm.at[pl.ds(s0 + c * CH, CH)], pebuf.at[slot],
            psem.at[slot]).wait()

    def x_load(c, slot):
        row = SC_B * S + s0 + c * CH
        pltpu.async_copy(
            x_hbm.at[pl.ds(row, CH)], xbuf.at[slot], xsem.at[slot])

    def x_wait(c, slot):
        row = SC_B * S + s0 + c * CH
        pltpu.make_async_copy(
            x_hbm.at[pl.ds(row, CH)], xbuf.at[slot], xsem.at[slot]).wait()

    def out_row(c):
        return s0 + c * CH

    def ostore_wait(c, slot):
        pltpu.make_async_copy(
            xbuf.at[slot], out_hbm.at[pl.ds(out_row(c), CH)],
            osem.at[slot]).wait()

    def step(t, k, skip_store_wait=False):
        x_wait(t, k)
        pe_wait(t, k % 2)
        xb = xbuf.at[k]
        pb = pebuf.at[k % 2]

        @plsc.parallel_loop(0, VECS, unroll=8)
        def _(j):
            r = lax.shift_right_logical(j, 6)
            o = pl.multiple_of(
                lax.shift_left(lax.bitwise_and(j, 63), 4), LANES)
            plsc.addupdate(
                xb.at[r].at[pl.ds(o, LANES)], pb.at[r][pl.ds(o, LANES)])

        pltpu.async_copy(
            xbuf.at[k], out_hbm.at[pl.ds(out_row(t), CH)], osem.at[k])

        if not skip_store_wait:
            ostore_wait(t, (k + 3) % 4)

        @pl.when(t + 3 < NCHUNK)
        def _():
            x_load(t + 3, (k + 3) % 4)

        @pl.when(t + 2 < NCHUNK)
        def _():
            pe_load(t + 2, k % 2)

    pe_load(0, 0)
    pe_load(1, 1)
    for k in range(3):
        x_load(k, k)
    step(0, 0, skip_store_wait=True)
    for k in range(1, 4):
        step(k, k)

    def body(g, carry):
        for k in range(4):
            step(4 * g + k, k)
        return carry

    lax.fori_loop(1, NCHUNK // 4, body, None)

    ostore_wait(NCHUNK - 1, (NCHUNK - 1) % 4)


def _tc_body(x_ref, pe_ref, o_ref):
    o_ref[...] = x_ref[...] + pe_ref[...]


def kernel(x, pe):
    x2 = x.reshape(B * S, D)
    sc_run = functools.partial(
        pl.kernel,
        mesh=plsc.VectorSubcoreMesh(core_axis_name="c", subcore_axis_name="s"),
        out_type=jax.ShapeDtypeStruct((S, D), jnp.float32),
        scratch_types=[
            pltpu.VMEM((4, CH, D), jnp.float32),
            pltpu.VMEM((2, CH, D), jnp.float32),
            pltpu.SemaphoreType.DMA((4,)),
            pltpu.SemaphoreType.DMA((2,)),
            pltpu.SemaphoreType.DMA((4,)),
        ],
    )(_sc_kernel)
    sc_out = sc_run(x2, pe)

    S_BLK = 2048
    ns = S // S_BLK
    tc_out = pl.pallas_call(
        _tc_body,
        grid=(ns, SC_B),
        in_specs=[
            pl.BlockSpec((1, S_BLK, D), lambda s, b: (b, s, 0)),
            pl.BlockSpec((S_BLK, D), lambda s, b: (s, 0)),
        ],
        out_specs=pl.BlockSpec((1, S_BLK, D), lambda s, b: (b, s, 0)),
        out_shape=jax.ShapeDtypeStruct((SC_B, S, D), x.dtype),
    )(x, pe)

    return jnp.concatenate(
        [tc_out, sc_out.reshape(1, S, D)], axis=0)


# speedup vs baseline: 2.2141x; 2.2141x over previous
import jax
import jax.numpy as jnp
from jax.experimental import pallas as pl
from jax.experimental.pallas import tpu as pltpu

B = 4
S = 8192
D = 1024
R = 1024
NS_BLK = S // R
NSTEP = NS_BLK * B
XSLOTS = 4
PSLOTS = 2


def _body(x_hbm, pe_hbm, out_hbm, xb, peb, xsem, psem, osem):
    def x_rows(t):
        s_blk, b = divmod(t, B)
        return b * S + s_blk * R

    def x_load(t):
        k = t % XSLOTS
        pltpu.make_async_copy(
            x_hbm.at[pl.ds(x_rows(t), R)], xb.at[k], xsem.at[k]).start()

    def x_wait(t):
        k = t % XSLOTS
        pltpu.make_async_copy(
            x_hbm.at[pl.ds(x_rows(t), R)], xb.at[k], xsem.at[k]).wait()

    def pe_load(s_blk):
        p = s_blk % PSLOTS
        pltpu.make_async_copy(
            pe_hbm.at[pl.ds(s_blk * R, R)], peb.at[p], psem.at[p]).start()

    def pe_wait(s_blk):
        p = s_blk % PSLOTS
        pltpu.make_async_copy(
            pe_hbm.at[pl.ds(s_blk * R, R)], peb.at[p], psem.at[p]).wait()

    def o_store(t):
        k = t % XSLOTS
        pltpu.make_async_copy(
            xb.at[k], out_hbm.at[pl.ds(x_rows(t), R)], osem.at[k]).start()

    def o_wait(t):
        k = t % XSLOTS
        pltpu.make_async_copy(
            xb.at[k], out_hbm.at[pl.ds(x_rows(t), R)], osem.at[k]).wait()

    pe_load(0)
    pe_load(1)
    for t in range(3):
        x_load(t)

    for t in range(NSTEP):
        s_blk, b = divmod(t, B)
        x_wait(t)
        if b == 0:
            pe_wait(s_blk)
        k = t % XSLOTS
        xb[k] = xb[k] + peb[s_blk % PSLOTS]
        o_store(t)
        if b == B - 1 and s_blk + 2 < NS_BLK:
            pe_load(s_blk + 2)
        if t >= 1:
            o_wait(t - 1)
        if t + 3 < NSTEP:
            x_load(t + 3)
    o_wait(NSTEP - 1)


def kernel(x, pe):
    x2 = x.reshape(B * S, D)
    out = pl.pallas_call(
        _body,
        in_specs=[
            pl.BlockSpec(memory_space=pl.ANY),
            pl.BlockSpec(memory_space=pl.ANY),
        ],
        out_specs=pl.BlockSpec(memory_space=pl.ANY),
        out_shape=jax.ShapeDtypeStruct((B * S, D), x.dtype),
        scratch_shapes=[
            pltpu.VMEM((XSLOTS, R, D), jnp.float32),
            pltpu.VMEM((PSLOTS, R, D), jnp.float32),
            pltpu.SemaphoreType.DMA((XSLOTS,)),
            pltpu.SemaphoreType.DMA((PSLOTS,)),
            pltpu.SemaphoreType.DMA((XSLOTS,)),
        ],
    )(x2, pe)
    return out.reshape(B, S, D)


# speedup vs baseline: 2.2253x; 1.0051x over previous
import jax
import jax.numpy as jnp
from jax.experimental import pallas as pl
from jax.experimental.pallas import tpu as pltpu

B = 4
S = 8192
D = 1024
R = 2048
NS_BLK = S // R
NSTEP = NS_BLK * B
XSLOTS = 4
PSLOTS = 2


def _body(x_hbm, pe_hbm, out_hbm, xb, peb, xsem, psem, osem):
    def x_rows(t):
        s_blk, b = divmod(t, B)
        return b * S + s_blk * R

    def x_load(t):
        k = t % XSLOTS
        pltpu.make_async_copy(
            x_hbm.at[pl.ds(x_rows(t), R)], xb.at[k], xsem.at[k]).start()

    def x_wait(t):
        k = t % XSLOTS
        pltpu.make_async_copy(
            x_hbm.at[pl.ds(x_rows(t), R)], xb.at[k], xsem.at[k]).wait()

    def pe_load(s_blk):
        p = s_blk % PSLOTS
        pltpu.make_async_copy(
            pe_hbm.at[pl.ds(s_blk * R, R)], peb.at[p], psem.at[p]).start()

    def pe_wait(s_blk):
        p = s_blk % PSLOTS
        pltpu.make_async_copy(
            pe_hbm.at[pl.ds(s_blk * R, R)], peb.at[p], psem.at[p]).wait()

    def o_store(t):
        k = t % XSLOTS
        pltpu.make_async_copy(
            xb.at[k], out_hbm.at[pl.ds(x_rows(t), R)], osem.at[k]).start()

    def o_wait(t):
        k = t % XSLOTS
        pltpu.make_async_copy(
            xb.at[k], out_hbm.at[pl.ds(x_rows(t), R)], osem.at[k]).wait()

    pe_load(0)
    pe_load(1)
    for t in range(3):
        x_load(t)

    for t in range(NSTEP):
        s_blk, b = divmod(t, B)
        x_wait(t)
        if b == 0:
            pe_wait(s_blk)
        k = t % XSLOTS
        xb[k] = xb[k] + peb[s_blk % PSLOTS]
        o_store(t)
        if b == B - 1 and s_blk + 2 < NS_BLK:
            pe_load(s_blk + 2)
        if t >= 1:
            o_wait(t - 1)
        if t + 3 < NSTEP:
            x_load(t + 3)
    o_wait(NSTEP - 1)


def kernel(x, pe):
    x2 = x.reshape(B * S, D)
    out = pl.pallas_call(
        _body,
        in_specs=[
            pl.BlockSpec(memory_space=pl.ANY),
            pl.BlockSpec(memory_space=pl.ANY),
        ],
        out_specs=pl.BlockSpec(memory_space=pl.ANY),
        out_shape=jax.ShapeDtypeStruct((B * S, D), x.dtype),
        scratch_shapes=[
            pltpu.VMEM((XSLOTS, R, D), jnp.float32),
            pltpu.VMEM((PSLOTS, R, D), jnp.float32),
            pltpu.SemaphoreType.DMA((XSLOTS,)),
            pltpu.SemaphoreType.DMA((PSLOTS,)),
            pltpu.SemaphoreType.DMA((XSLOTS,)),
        ],
    )(x2, pe)
    return out.reshape(B, S, D)
